# baseline (speedup 1.0000x reference)
import functools
import math

import jax
import jax.numpy as jnp
from jax import lax
from jax.experimental import pallas as pl
from jax.experimental.pallas import tpu as pltpu
from jax.experimental.pallas import tpu_sc as plsc

N = 10000
E = 160000
D = 256
H = 128
C = 10
G = 8
EPS = 1e-5

NSUB = 16
K = 128
CH = 80
CHC = 40
CHH = 40
EPT = CH * K
EPAD = EPT * NSUB
ZROWS = 632
NACC = ZROWS * NSUB
WRT = 624
WTAIL = N - WRT * NSUB
BLK = 1000
BPAD = 10240


def _make_sc_kernel(count_mode):
    mesh = plsc.VectorSubcoreMesh(core_axis_name="c", subcore_axis_name="s")
    nch = CHC if count_mode else CH

    def body(*refs):
        if count_mode:
            dst2_hbm, out_hbm, didx, rows, acc, sem = refs
        else:
            (x_hbm, src2_hbm, dstp_hbm, out_hbm, sidx, didx, rows, rows1,
             acc, sem, sem1, ssem, ssem1) = refs
        c = lax.axis_index("c")
        s = lax.axis_index("s")
        if count_mode:
            pltpu.sync_copy(dst2_hbm.at[c, s], didx)
        fill = jnp.full((16,), 1.0 if count_mode else 0.0, jnp.float32)
        zero = jnp.zeros((16,), jnp.float32)

        def frow(i, carry):
            for k2 in range(8):
                rows[i, pl.ds(k2 * 16, 16)] = zero
            return carry

        lax.fori_loop(0, K, frow, 0)
        for zo, zn in ((0, 128), (128, 128), (256, 128), (384, 128),
                       (512, ZROWS - 512)):
            pltpu.sync_copy(rows.at[pl.ds(0, zn)],
                            acc.at[pl.ds(s * ZROWS + zo, zn)])
        if count_mode:
            def orow(i, carry):
                for k2 in range(8):
                    rows[i, pl.ds(k2 * 16, 16)] = fill
                return carry

            lax.fori_loop(0, K, orow, 0)
        plsc.subcore_barrier()

        if count_mode:
            def chunk(j, carry):
                pltpu.sync_copy(rows, acc.at[didx.at[j]], add=True)
                return carry

            lax.fori_loop(0, nch, chunk, 0)
        else:
            nout = CHH // 2

            def half(hh, carry):
                pltpu.sync_copy(src2_hbm.at[c, s, pl.ds(hh * CHH, CHH)], sidx)
                pltpu.sync_copy(dstp_hbm.at[s, pl.ds(hh * CHH, CHH)], didx)
                pltpu.async_copy(x_hbm.at[sidx.at[0]], rows, sem)
                pltpu.async_copy(x_hbm.at[sidx.at[1]], rows1, sem1)

                def outer(jj, carry2):
                    j0 = 2 * jj
                    j1 = j0 + 1
                    pltpu.make_async_copy(x_hbm.at[sidx.at[j0]], rows,
                                          sem).wait()
                    pltpu.async_copy(rows, acc.at[didx.at[j0]], ssem,
                                     add=True)
                    pltpu.make_async_copy(x_hbm.at[sidx.at[j1]], rows1,
                                          sem1).wait()
                    pltpu.async_copy(rows1, acc.at[didx.at[j1]], ssem1,
                                     add=True)

                    @pl.when(jj < nout - 1)
                    def _():
                        pltpu.make_async_copy(rows, acc.at[didx.at[j0]],
                                              ssem).wait()
                        pltpu.async_copy(x_hbm.at[sidx.at[j0 + 2]], rows, sem)
                        pltpu.make_async_copy(rows1, acc.at[didx.at[j1]],
                                              ssem1).wait()
                        pltpu.async_copy(x_hbm.at[sidx.at[j1 + 2]], rows1,
                                         sem1)
                    return carry2

                lax.fori_loop(0, nout, outer, 0)
                pltpu.make_async_copy(rows, acc.at[didx.at[CHH - 2]],
                                      ssem).wait()
                pltpu.make_async_copy(rows1, acc.at[didx.at[CHH - 1]],
                                      ssem1).wait()
                return carry

            lax.fori_loop(0, CH // CHH, half, 0)
        plsc.subcore_barrier()
        base = s * WRT

        def wout(j, carry):
            wo = j * K
            pltpu.sync_copy(acc.at[pl.ds(base + wo, K)], rows)
            pltpu.sync_copy(rows, out_hbm.at[pl.ds(c * N + base + wo, K)])
            return carry

        lax.fori_loop(0, WRT // K, wout, 0)
        wt = WRT - (WRT // K) * K
        wo2 = (WRT // K) * K
        pltpu.sync_copy(acc.at[pl.ds(base + wo2, wt)], rows.at[pl.ds(0, wt)])
        pltpu.sync_copy(rows.at[pl.ds(0, wt)],
                        out_hbm.at[pl.ds(c * N + base + wo2, wt)])

        @pl.when(s == NSUB - 1)
        def _():
            tb = WRT * NSUB
            pltpu.sync_copy(acc.at[pl.ds(tb, WTAIL)], rows.at[pl.ds(0, WTAIL)])
            pltpu.sync_copy(rows.at[pl.ds(0, WTAIL)],
                            out_hbm.at[pl.ds(c * N + tb, WTAIL)])

    if count_mode:
        scratch = [
            pltpu.VMEM((nch, K), jnp.int32),
            pltpu.VMEM((K, H), jnp.float32),
            pltpu.VMEM_SHARED((NACC, H), jnp.float32),
            pltpu.SemaphoreType.DMA,
        ]
    else:
        scratch = [
            pltpu.VMEM((CHH, K), jnp.int32),
            pltpu.VMEM((CHH, K), jnp.int32),
            pltpu.VMEM((K, H), jnp.float32),
            pltpu.VMEM((K, H), jnp.float32),
            pltpu.VMEM_SHARED((NACC, H), jnp.float32),
            pltpu.SemaphoreType.DMA,
            pltpu.SemaphoreType.DMA,
            pltpu.SemaphoreType.DMA,
            pltpu.SemaphoreType.DMA,
        ]
    return pl.kernel(
        body,
        out_type=jax.ShapeDtypeStruct((2 * N, H), jnp.float32),
        mesh=mesh,
        scratch_types=scratch,
    )


def _dense1_body(agg_ref, cnt_ref, x_ref, wl_ref, wc_ref, sc_ref, sh_ref,
                 o_ref):
    inv = 1.0 / jnp.maximum(cnt_ref[0][:, 0:1] + cnt_ref[1][:, 0:1], 1.0)
    mean = jnp.concatenate([agg_ref[0], agg_ref[1]], axis=1) * inv
    xf = jnp.concatenate([x_ref[0], x_ref[1]], axis=1)
    y = (jnp.dot(mean, wl_ref[...], preferred_element_type=jnp.float32)
         + jnp.dot(xf, wc_ref[...], preferred_element_type=jnp.float32))
    out = jnp.maximum(y * sc_ref[...] + sh_ref[...], 0.0)
    o_ref[0] = out[:, :H]
    o_ref[1] = out[:, H:]


def _dense2_body(agg_ref, cnt_ref, h_ref, wl_ref, wc_ref, sc_ref, sh_ref,
                 wih_ref, bias_ref, z_ref):
    inv = 1.0 / jnp.maximum(cnt_ref[0][:, 0:1] + cnt_ref[1][:, 0:1], 1.0)
    mean = jnp.concatenate([agg_ref[0], agg_ref[1]], axis=1) * inv
    hf = jnp.concatenate([h_ref[0], h_ref[1]], axis=1)
    y = (jnp.dot(mean, wl_ref[...], preferred_element_type=jnp.float32)
         + jnp.dot(hf, wc_ref[...], preferred_element_type=jnp.float32))
    h2 = jnp.maximum(y * sc_ref[...] + sh_ref[...], 0.0)
    z_ref[...] = (jnp.dot(h2, wih_ref[...], preferred_element_type=jnp.float32)
                  + bias_ref[...])


def _lstm_body(z_ref, batch_ref, whh_ref, w1_ref, b1_ref, w2_ref, b2_ref,
               out_ref):
    bt = batch_ref[...]
    lens = []
    starts = []
    acc = jnp.int32(0)
    for g in range(G):
        lg = jnp.sum(jnp.where(bt == g, 1, 0).astype(jnp.int32))
        starts.append(acc)
        lens.append(lg)
        acc = acc + lg
    maxlen = lens[0]
    for g in range(1, G):
        maxlen = jnp.maximum(maxlen, lens[g])
    lenmat = jnp.concatenate(
        [jnp.full((1, H), lens[g], jnp.int32) for g in range(G)], axis=0)

    def load_xt(t):
        rows = []
        for g in range(G):
            idx = starts[g] + jnp.minimum(t, lens[g] - 1)
            idx = jnp.minimum(jnp.maximum(idx, 0), N - 1)
            rows.append(z_ref[pl.ds(idx, 1), :])
        return jnp.concatenate(rows, axis=0)

    def step(t, carry):
        h, c, xt = carry
        zt = xt + jnp.dot(h, whh_ref[...], preferred_element_type=jnp.float32)
        ig = jax.nn.sigmoid(zt[:, 0:H])
        fg = jax.nn.sigmoid(zt[:, H:2 * H])
        gg = jnp.tanh(zt[:, 2 * H:3 * H])
        og = jax.nn.sigmoid(zt[:, 3 * H:4 * H])
        cn = fg * c + ig * gg
        hn = og * jnp.tanh(cn)
        mask = lenmat > t
        h = jnp.where(mask, hn, h)
        c = jnp.where(mask, cn, c)
        return (h, c, load_xt(t + 1))

    h0 = jnp.zeros((G, H), jnp.float32)
    c0 = jnp.zeros((G, H), jnp.float32)
    h, _, _ = lax.fori_loop(0, maxlen, step,
                            (h0, c0, load_xt(jnp.int32(0))))
    hid = jnp.maximum(
        jnp.dot(h, w1_ref[...], preferred_element_type=jnp.float32)
        + b1_ref[...], 0.0)
    out_ref[...] = (jnp.dot(hid, w2_ref[...], preferred_element_type=jnp.float32)
                    + b2_ref[...])


def kernel(x, edge_index, batch, Wl0, bl0, Wr0, Wres0, g0, be0, Wl1, bl1, Wr1,
           Wres1, g1, be1, W_ih, W_hh, b_ih, b_hh, W1, b1, W2, b2):
    f32 = jnp.float32
    src = edge_index[0]
    dst = edge_index[1]
    pad = EPAD - E
    srcp = jnp.concatenate([src, jnp.zeros((pad,), jnp.int32)])
    src2 = jnp.stack([srcp, srcp + N]).reshape(2, NSUB, CH, K)
    dstp = jnp.concatenate([dst, jnp.full((pad,), N, jnp.int32)]
                           ).reshape(NSUB, CH, K)
    x2 = x.reshape(N, 2, H).transpose(1, 0, 2).reshape(2 * N, H)

    dst2 = dstp.reshape(2, NSUB, CHC, K)

    agg = _make_sc_kernel(False)
    cntk = _make_sc_kernel(True)

    cnt = cntk(dst2).reshape(2, N, H)
    agg0 = agg(x2, src2, dstp)

    rs = 1.0 / math.sqrt(1.0 + EPS)
    scale0 = (g0 * rs).reshape(1, D)
    shift0 = (g0 * rs * bl0 + be0).reshape(1, D)
    scale1 = (g1 * rs).reshape(1, D)
    shift1 = (g1 * rs * bl1 + be1).reshape(1, D)
    Wl0T = Wl0.T
    Wc0T = (Wr0 + Wres0).T
    Wl1T = Wl1.T
    Wc1T = (Wr1 + Wres1).T
    WihT = W_ih.T
    bias2 = (b_ih + b_hh).reshape(1, 4 * H)
    WhhT = W_hh.T
    W1T = W1.T
    W2T = W2.T

    nblk = N // BLK
    split_spec = pl.BlockSpec((2, BLK, H), lambda i: (0, i, 0))
    cnt_spec = split_spec
    w_spec = pl.BlockSpec((D, D), lambda i: (0, 0))
    v_spec = pl.BlockSpec((1, D), lambda i: (0, 0))

    h1 = pl.pallas_call(
        _dense1_body,
        grid=(nblk,),
        in_specs=[split_spec, cnt_spec, split_spec, w_spec, w_spec, v_spec,
                  v_spec],
        out_specs=split_spec,
        out_shape=jax.ShapeDtypeStruct((2, N, H), f32),
    )(agg0.reshape(2, N, H), cnt, x2.reshape(2, N, H), Wl0T, Wc0T, scale0,
      shift0)

    agg1 = agg(h1.reshape(2 * N, H), src2, dstp)

    z = pl.pallas_call(
        _dense2_body,
        grid=(nblk,),
        in_specs=[split_spec, cnt_spec, split_spec, w_spec, w_spec, v_spec,
                  v_spec, pl.BlockSpec((D, 4 * H), lambda i: (0, 0)),
                  pl.BlockSpec((1, 4 * H), lambda i: (0, 0))],
        out_specs=pl.BlockSpec((BLK, 4 * H), lambda i: (i, 0)),
        out_shape=jax.ShapeDtypeStruct((N, 4 * H), f32),
    )(agg1.reshape(2, N, H), cnt, h1.reshape(2, N, H), Wl1T, Wc1T, scale1,
      shift1, WihT, bias2)

    batch_pad = jnp.concatenate(
        [batch, jnp.full((BPAD - N,), G, jnp.int32)]).reshape(BPAD // 128, 128)

    out = pl.pallas_call(
        _lstm_body,
        out_shape=jax.ShapeDtypeStruct((G, C), f32),
    )(z, batch_pad, WhhT, W1T, b1.reshape(1, 64), W2T, b2.reshape(1, C))
    return out


# speedup vs baseline: 1.0520x; 1.0520x over previous
import functools
import math

import jax
import jax.numpy as jnp
from jax import lax
from jax.experimental import pallas as pl
from jax.experimental.pallas import tpu as pltpu
from jax.experimental.pallas import tpu_sc as plsc

N = 10000
E = 160000
D = 256
H = 128
C = 10
G = 8
EPS = 1e-5

NSUB = 16
K = 128
CH = 80
CHC = 40
CHH = 40
EPT = CH * K
EPAD = EPT * NSUB
ZROWS = 632
NACC = ZROWS * NSUB
WRT = 624
WTAIL = N - WRT * NSUB
BLK = 1000
BPAD = 10240


def _make_sc_kernel(count_mode):
    mesh = plsc.VectorSubcoreMesh(core_axis_name="c", subcore_axis_name="s")
    nch = CHC if count_mode else CH

    def body(*refs):
        if count_mode:
            dst2_hbm, out_hbm, didx, rows, acc, sem = refs
        else:
            (x_hbm, src2_hbm, dstp_hbm, out_hbm, sidx, didx, rows, rows1,
             acc, sem, sem1) = refs
        c = lax.axis_index("c")
        s = lax.axis_index("s")
        if count_mode:
            pltpu.sync_copy(dst2_hbm.at[c, s], didx)
        fill = jnp.full((16,), 1.0 if count_mode else 0.0, jnp.float32)
        zero = jnp.zeros((16,), jnp.float32)

        def frow(i, carry):
            for k2 in range(8):
                rows[i, pl.ds(k2 * 16, 16)] = zero
            return carry

        lax.fori_loop(0, K, frow, 0)
        for zo, zn in ((0, 128), (128, 128), (256, 128), (384, 128),
                       (512, ZROWS - 512)):
            pltpu.sync_copy(rows.at[pl.ds(0, zn)],
                            acc.at[pl.ds(s * ZROWS + zo, zn)])
        if count_mode:
            def orow(i, carry):
                for k2 in range(8):
                    rows[i, pl.ds(k2 * 16, 16)] = fill
                return carry

            lax.fori_loop(0, K, orow, 0)
        plsc.subcore_barrier()

        if count_mode:
            def chunk(j, carry):
                pltpu.sync_copy(rows, acc.at[didx.at[j]], add=True)
                return carry

            lax.fori_loop(0, nch, chunk, 0)
        else:
            nout = CHH // 2

            def half(hh, carry):
                pltpu.sync_copy(src2_hbm.at[c, s, pl.ds(hh * CHH, CHH)], sidx)
                pltpu.sync_copy(dstp_hbm.at[s, pl.ds(hh * CHH, CHH)], didx)
                pltpu.async_copy(x_hbm.at[sidx.at[0]], rows, sem)

                def outer(jj, carry2):
                    j0 = 2 * jj
                    j1 = j0 + 1
                    pltpu.async_copy(x_hbm.at[sidx.at[j1]], rows1, sem1)
                    pltpu.make_async_copy(x_hbm.at[sidx.at[j0]], rows,
                                          sem).wait()
                    pltpu.sync_copy(rows, acc.at[didx.at[j0]], add=True)

                    @pl.when(jj < nout - 1)
                    def _():
                        pltpu.async_copy(x_hbm.at[sidx.at[j0 + 2]], rows, sem)

                    pltpu.make_async_copy(x_hbm.at[sidx.at[j1]], rows1,
                                          sem1).wait()
                    pltpu.sync_copy(rows1, acc.at[didx.at[j1]], add=True)
                    return carry2

                lax.fori_loop(0, nout, outer, 0)
                return carry

            lax.fori_loop(0, CH // CHH, half, 0)
        plsc.subcore_barrier()
        base = s * WRT

        def wout(j, carry):
            wo = j * K
            pltpu.sync_copy(acc.at[pl.ds(base + wo, K)], rows)
            pltpu.sync_copy(rows, out_hbm.at[pl.ds(c * N + base + wo, K)])
            return carry

        lax.fori_loop(0, WRT // K, wout, 0)
        wt = WRT - (WRT // K) * K
        wo2 = (WRT // K) * K
        pltpu.sync_copy(acc.at[pl.ds(base + wo2, wt)], rows.at[pl.ds(0, wt)])
        pltpu.sync_copy(rows.at[pl.ds(0, wt)],
                        out_hbm.at[pl.ds(c * N + base + wo2, wt)])

        @pl.when(s == NSUB - 1)
        def _():
            tb = WRT * NSUB
            pltpu.sync_copy(acc.at[pl.ds(tb, WTAIL)], rows.at[pl.ds(0, WTAIL)])
            pltpu.sync_copy(rows.at[pl.ds(0, WTAIL)],
                            out_hbm.at[pl.ds(c * N + tb, WTAIL)])

    if count_mode:
        scratch = [
            pltpu.VMEM((nch, K), jnp.int32),
            pltpu.VMEM((K, H), jnp.float32),
            pltpu.VMEM_SHARED((NACC, H), jnp.float32),
            pltpu.SemaphoreType.DMA,
        ]
    else:
        scratch = [
            pltpu.VMEM((CHH, K), jnp.int32),
            pltpu.VMEM((CHH, K), jnp.int32),
            pltpu.VMEM((K, H), jnp.float32),
            pltpu.VMEM((K, H), jnp.float32),
            pltpu.VMEM_SHARED((NACC, H), jnp.float32),
            pltpu.SemaphoreType.DMA,
            pltpu.SemaphoreType.DMA,
        ]
    return pl.kernel(
        body,
        out_type=jax.ShapeDtypeStruct((2 * N, H), jnp.float32),
        mesh=mesh,
        scratch_types=scratch,
    )


def _dense1_body(agg_ref, cnt_ref, x_ref, wl_ref, wc_ref, sc_ref, sh_ref,
                 o_ref):
    inv = 1.0 / jnp.maximum(cnt_ref[0][:, 0:1] + cnt_ref[1][:, 0:1], 1.0)
    mean = jnp.concatenate([agg_ref[0], agg_ref[1]], axis=1) * inv
    xf = jnp.concatenate([x_ref[0], x_ref[1]], axis=1)
    y = (jnp.dot(mean, wl_ref[...], preferred_element_type=jnp.float32)
         + jnp.dot(xf, wc_ref[...], preferred_element_type=jnp.float32))
    out = jnp.maximum(y * sc_ref[...] + sh_ref[...], 0.0)
    o_ref[0] = out[:, :H]
    o_ref[1] = out[:, H:]


def _dense2_body(agg_ref, cnt_ref, h_ref, wl_ref, wc_ref, sc_ref, sh_ref,
                 wih_ref, bias_ref, z_ref):
    inv = 1.0 / jnp.maximum(cnt_ref[0][:, 0:1] + cnt_ref[1][:, 0:1], 1.0)
    mean = jnp.concatenate([agg_ref[0], agg_ref[1]], axis=1) * inv
    hf = jnp.concatenate([h_ref[0], h_ref[1]], axis=1)
    y = (jnp.dot(mean, wl_ref[...], preferred_element_type=jnp.float32)
         + jnp.dot(hf, wc_ref[...], preferred_element_type=jnp.float32))
    h2 = jnp.maximum(y * sc_ref[...] + sh_ref[...], 0.0)
    z_ref[...] = (jnp.dot(h2, wih_ref[...], preferred_element_type=jnp.float32)
                  + bias_ref[...])


def _lstm_body(z_ref, batch_ref, whh_ref, w1_ref, b1_ref, w2_ref, b2_ref,
               out_ref):
    bt = batch_ref[...]
    lens = []
    starts = []
    acc = jnp.int32(0)
    for g in range(G):
        lg = jnp.sum(jnp.where(bt == g, 1, 0).astype(jnp.int32))
        starts.append(acc)
        lens.append(lg)
        acc = acc + lg
    maxlen = lens[0]
    for g in range(1, G):
        maxlen = jnp.maximum(maxlen, lens[g])
    lenmat = jnp.concatenate(
        [jnp.full((1, H), lens[g], jnp.int32) for g in range(G)], axis=0)

    def load_xt(t):
        rows = []
        for g in range(G):
            idx = starts[g] + jnp.minimum(t, lens[g] - 1)
            idx = jnp.minimum(jnp.maximum(idx, 0), N - 1)
            rows.append(z_ref[pl.ds(idx, 1), :])
        return jnp.concatenate(rows, axis=0)

    def step(t, carry):
        h, c, xt = carry
        zt = xt + jnp.dot(h, whh_ref[...], preferred_element_type=jnp.float32)
        ig = jax.nn.sigmoid(zt[:, 0:H])
        fg = jax.nn.sigmoid(zt[:, H:2 * H])
        gg = jnp.tanh(zt[:, 2 * H:3 * H])
        og = jax.nn.sigmoid(zt[:, 3 * H:4 * H])
        cn = fg * c + ig * gg
        hn = og * jnp.tanh(cn)
        mask = lenmat > t
        h = jnp.where(mask, hn, h)
        c = jnp.where(mask, cn, c)
        return (h, c, load_xt(t + 1))

    h0 = jnp.zeros((G, H), jnp.float32)
    c0 = jnp.zeros((G, H), jnp.float32)
    h, _, _ = lax.fori_loop(0, maxlen, step,
                            (h0, c0, load_xt(jnp.int32(0))))
    hid = jnp.maximum(
        jnp.dot(h, w1_ref[...], preferred_element_type=jnp.float32)
        + b1_ref[...], 0.0)
    out_ref[...] = (jnp.dot(hid, w2_ref[...], preferred_element_type=jnp.float32)
                    + b2_ref[...])


def kernel(x, edge_index, batch, Wl0, bl0, Wr0, Wres0, g0, be0, Wl1, bl1, Wr1,
           Wres1, g1, be1, W_ih, W_hh, b_ih, b_hh, W1, b1, W2, b2):
    f32 = jnp.float32
    src = edge_index[0]
    dst = edge_index[1]
    pad = EPAD - E
    srcp = jnp.concatenate([src, jnp.zeros((pad,), jnp.int32)])
    src2 = jnp.stack([srcp, srcp + N]).reshape(2, NSUB, CH, K)
    dstp = jnp.concatenate([dst, jnp.full((pad,), N, jnp.int32)]
                           ).reshape(NSUB, CH, K)
    x2 = x.reshape(N, 2, H).transpose(1, 0, 2).reshape(2 * N, H)

    dst2 = dstp.reshape(2, NSUB, CHC, K)

    agg = _make_sc_kernel(False)
    cntk = _make_sc_kernel(True)

    cnt = cntk(dst2).reshape(2, N, H)
    agg0 = agg(x2, src2, dstp)

    rs = 1.0 / math.sqrt(1.0 + EPS)
    scale0 = (g0 * rs).reshape(1, D)
    shift0 = (g0 * rs * bl0 + be0).reshape(1, D)
    scale1 = (g1 * rs).reshape(1, D)
    shift1 = (g1 * rs * bl1 + be1).reshape(1, D)
    Wl0T = Wl0.T
    Wc0T = (Wr0 + Wres0).T
    Wl1T = Wl1.T
    Wc1T = (Wr1 + Wres1).T
    WihT = W_ih.T
    bias2 = (b_ih + b_hh).reshape(1, 4 * H)
    WhhT = W_hh.T
    W1T = W1.T
    W2T = W2.T

    nblk = N // BLK
    split_spec = pl.BlockSpec((2, BLK, H), lambda i: (0, i, 0))
    cnt_spec = split_spec
    w_spec = pl.BlockSpec((D, D), lambda i: (0, 0))
    v_spec = pl.BlockSpec((1, D), lambda i: (0, 0))

    h1 = pl.pallas_call(
        _dense1_body,
        grid=(nblk,),
        in_specs=[split_spec, cnt_spec, split_spec, w_spec, w_spec, v_spec,
                  v_spec],
        out_specs=split_spec,
        out_shape=jax.ShapeDtypeStruct((2, N, H), f32),
    )(agg0.reshape(2, N, H), cnt, x2.reshape(2, N, H), Wl0T, Wc0T, scale0,
      shift0)

    agg1 = agg(h1.reshape(2 * N, H), src2, dstp)

    z = pl.pallas_call(
        _dense2_body,
        grid=(nblk,),
        in_specs=[split_spec, cnt_spec, split_spec, w_spec, w_spec, v_spec,
                  v_spec, pl.BlockSpec((D, 4 * H), lambda i: (0, 0)),
                  pl.BlockSpec((1, 4 * H), lambda i: (0, 0))],
        out_specs=pl.BlockSpec((BLK, 4 * H), lambda i: (i, 0)),
        out_shape=jax.ShapeDtypeStruct((N, 4 * H), f32),
    )(agg1.reshape(2, N, H), cnt, h1.reshape(2, N, H), Wl1T, Wc1T, scale1,
      shift1, WihT, bias2)

    batch_pad = jnp.concatenate(
        [batch, jnp.full((BPAD - N,), G, jnp.int32)]).reshape(BPAD // 128, 128)

    out = pl.pallas_call(
        _lstm_body,
        out_shape=jax.ShapeDtypeStruct((G, C), f32),
    )(z, batch_pad, WhhT, W1T, b1.reshape(1, 64), W2T, b2.reshape(1, C))
    return out
